# Initial kernel scaffold; baseline (speedup 1.0000x reference)
import jax
import jax.numpy as jnp
from jax.experimental import pallas as pl


def kernel(x, w1, b1, w2, b2):
    raise NotImplementedError("write your pallas kernel here")


# speedup vs baseline: 1.4637x; 1.4637x over previous
import jax
import jax.numpy as jnp
from jax.experimental import pallas as pl
from jax.experimental.pallas import tpu as pltpu


def _make_fused_body(HW):
    inv_hw = 1.0 / float(HW)
    lanes_ok = (HW % 128 == 0)
    n_strips = HW // 128 if lanes_ok else 1

    def _body(x_ref, w1_ref, b1_ref, w2_ref, b2_ref, o_ref):
        x = x_ref[0]

        if lanes_ok:
            part = x[:, 0:128]
            for i in range(1, n_strips):
                part = part + x[:, i * 128:(i + 1) * 128]
            p = jnp.sum(part, axis=-1, keepdims=True) * inv_hw
        else:
            p = jnp.sum(x, axis=-1, keepdims=True) * inv_hw

        h = jnp.dot(w1_ref[...], p, preferred_element_type=jnp.float32)
        h = jnp.maximum(h + b1_ref[...], 0.0)
        z = jnp.dot(w2_ref[...], h, preferred_element_type=jnp.float32)
        s = jax.nn.sigmoid(z + b2_ref[...])

        o_ref[0] = x * s

    return _body


def kernel(x, w1, b1, w2, b2):
    B, C, H, W = x.shape
    Cr = w1.shape[0]
    HW = H * W
    x_flat = x.reshape(B, C, HW)
    b1c = b1.reshape(Cr, 1)
    b2c = b2.reshape(C, 1)

    out = pl.pallas_call(
        _make_fused_body(HW),
        out_shape=jax.ShapeDtypeStruct((B, C, HW), jnp.float32),
        grid_spec=pltpu.PrefetchScalarGridSpec(
            num_scalar_prefetch=0,
            grid=(B,),
            in_specs=[
                pl.BlockSpec((1, C, HW), lambda b: (b, 0, 0)),
                pl.BlockSpec((Cr, C), lambda b: (0, 0)),
                pl.BlockSpec((Cr, 1), lambda b: (0, 0)),
                pl.BlockSpec((C, Cr), lambda b: (0, 0)),
                pl.BlockSpec((C, 1), lambda b: (0, 0)),
            ],
            out_specs=pl.BlockSpec((1, C, HW), lambda b: (b, 0, 0)),
        ),
        compiler_params=pltpu.CompilerParams(
            dimension_semantics=("parallel",)),
    )(x_flat, w1, b1c, w2, b2c)

    return out.reshape(B, C, H, W)


# speedup vs baseline: 1.5415x; 1.0532x over previous
import jax
import jax.numpy as jnp
from jax.experimental import pallas as pl
from jax.experimental.pallas import tpu as pltpu


def _make_fused_body(HW, BB):
    inv_hw = 1.0 / float(HW)
    lanes_ok = (HW % 128 == 0)
    n_strips = HW // 128 if lanes_ok else 1

    def _body(x_ref, w1_ref, b1_ref, w2_ref, b2_ref, o_ref):
        for bi in range(BB):
            x = x_ref[bi]

            if lanes_ok:
                part = x[:, 0:128]
                for i in range(1, n_strips):
                    part = part + x[:, i * 128:(i + 1) * 128]
                p = jnp.sum(part, axis=-1, keepdims=True) * inv_hw
            else:
                p = jnp.sum(x, axis=-1, keepdims=True) * inv_hw

            h = jnp.dot(w1_ref[...], p, preferred_element_type=jnp.float32)
            h = jnp.maximum(h + b1_ref[...], 0.0)
            z = jnp.dot(w2_ref[...], h, preferred_element_type=jnp.float32)
            s = jax.nn.sigmoid(z + b2_ref[...])

            o_ref[bi] = x * s

    return _body


def kernel(x, w1, b1, w2, b2):
    B, C, H, W = x.shape
    Cr = w1.shape[0]
    HW = H * W
    x_flat = x.reshape(B, C, HW)
    b1c = b1.reshape(Cr, 1)
    b2c = b2.reshape(C, 1)

    BB = 2 if (B % 2 == 0) else 1

    out = pl.pallas_call(
        _make_fused_body(HW, BB),
        out_shape=jax.ShapeDtypeStruct((B, C, HW), jnp.float32),
        grid_spec=pltpu.PrefetchScalarGridSpec(
            num_scalar_prefetch=0,
            grid=(B // BB,),
            in_specs=[
                pl.BlockSpec((BB, C, HW), lambda b: (b, 0, 0)),
                pl.BlockSpec((Cr, C), lambda b: (0, 0)),
                pl.BlockSpec((Cr, 1), lambda b: (0, 0)),
                pl.BlockSpec((C, Cr), lambda b: (0, 0)),
                pl.BlockSpec((C, 1), lambda b: (0, 0)),
            ],
            out_specs=pl.BlockSpec((BB, C, HW), lambda b: (b, 0, 0)),
        ),
        compiler_params=pltpu.CompilerParams(
            dimension_semantics=("parallel",)),
    )(x_flat, w1, b1c, w2, b2c)

    return out.reshape(B, C, H, W)


# speedup vs baseline: 1.5657x; 1.0157x over previous
import jax
import jax.numpy as jnp
from jax.experimental import pallas as pl
from jax.experimental.pallas import tpu as pltpu


def _make_fused_body(HW, BB):
    inv_hw = 1.0 / float(HW)
    lanes_ok = (HW % 128 == 0)
    n_strips = HW // 128 if lanes_ok else 1

    def _body(x_ref, w1_ref, b1_ref, w2_ref, b2_ref, o_ref):
        for bi in range(BB):
            x = x_ref[bi]

            if lanes_ok:
                part = x[:, 0:128]
                for i in range(1, n_strips):
                    part = part + x[:, i * 128:(i + 1) * 128]
                p = jnp.sum(part, axis=-1, keepdims=True) * inv_hw
            else:
                p = jnp.sum(x, axis=-1, keepdims=True) * inv_hw

            h = jnp.dot(w1_ref[...], p, preferred_element_type=jnp.float32)
            h = jnp.maximum(h + b1_ref[...], 0.0)
            z = jnp.dot(w2_ref[...], h, preferred_element_type=jnp.float32)
            s = jax.nn.sigmoid(z + b2_ref[...])

            o_ref[bi] = x * s

    return _body


def kernel(x, w1, b1, w2, b2):
    B, C, H, W = x.shape
    Cr = w1.shape[0]
    HW = H * W
    x_flat = x.reshape(B, C, HW)
    b1c = b1.reshape(Cr, 1)
    b2c = b2.reshape(C, 1)

    BB = 4 if (B % 4 == 0) else (2 if (B % 2 == 0) else 1)

    out = pl.pallas_call(
        _make_fused_body(HW, BB),
        out_shape=jax.ShapeDtypeStruct((B, C, HW), jnp.float32),
        grid_spec=pltpu.PrefetchScalarGridSpec(
            num_scalar_prefetch=0,
            grid=(B // BB,),
            in_specs=[
                pl.BlockSpec((BB, C, HW), lambda b: (b, 0, 0)),
                pl.BlockSpec((Cr, C), lambda b: (0, 0)),
                pl.BlockSpec((Cr, 1), lambda b: (0, 0)),
                pl.BlockSpec((C, Cr), lambda b: (0, 0)),
                pl.BlockSpec((C, 1), lambda b: (0, 0)),
            ],
            out_specs=pl.BlockSpec((BB, C, HW), lambda b: (b, 0, 0)),
        ),
        compiler_params=pltpu.CompilerParams(
            dimension_semantics=("parallel",)),
    )(x_flat, w1, b1c, w2, b2c)

    return out.reshape(B, C, H, W)
